# baseline (speedup 1.0000x reference)
import functools

import jax
import jax.numpy as jnp
from jax import lax
from jax.experimental import pallas as pl
from jax.experimental.pallas import tpu as pltpu
from jax.experimental.pallas import tpu_sc as plsc

B, H, D = 16384, 200, 64
N = B * H
NC, NS = 2, 16
NW = NC * NS
PER_W = N // NW
CHUNK = 640
SUB = 640
STEPS = PER_W // CHUNK
NBUF = 2
NGROUPS = STEPS // NBUF

_mesh = plsc.VectorSubcoreMesh(core_axis_name="c", subcore_axis_name="s")


@functools.partial(
    pl.kernel,
    mesh=_mesh,
    out_type=jax.ShapeDtypeStruct((N, D), jnp.float32),
    scratch_types=[
        pltpu.VMEM((CHUNK,), jnp.int32),
        pltpu.VMEM((CHUNK,), jnp.int32),
        pltpu.VMEM((CHUNK, D), jnp.float32),
        pltpu.VMEM((CHUNK, D), jnp.float32),
        pltpu.SemaphoreType.DMA,
        pltpu.SemaphoreType.DMA,
        pltpu.SemaphoreType.DMA,
        pltpu.SemaphoreType.DMA,
    ],
    compiler_params=pltpu.CompilerParams(use_tc_tiling_on_sc=False),
)
def _embed(idx_hbm, table_hbm, out_hbm, idx0, idx1, rows0, rows1,
           gsem0, gsem1, osem0, osem1):
    idx_v = [idx0, idx1]
    rows_v = [rows0, rows1]
    gsems = [gsem0, gsem1]
    osems = [osem0, osem1]

    wid = lax.axis_index("s") * NC + lax.axis_index("c")
    base = wid * PER_W

    def fire_gather(chunk_id, b):
        off = base + chunk_id * CHUNK
        pltpu.sync_copy(idx_hbm.at[pl.ds(off, CHUNK)], idx_v[b])
        for j in range(CHUNK // SUB):
            pltpu.async_copy(
                table_hbm.at[idx_v[b].at[pl.ds(j * SUB, SUB)]],
                rows_v[b].at[pl.ds(j * SUB, SUB)],
                gsems[b],
            )

    def wait_gathers(b):
        pltpu.make_async_copy(
            out_hbm.at[pl.ds(0, CHUNK)], rows_v[b], gsems[b]
        ).wait()

    def fire_store(chunk_id, b):
        off = base + chunk_id * CHUNK
        pltpu.async_copy(rows_v[b], out_hbm.at[pl.ds(off, CHUNK)], osems[b])

    def wait_store(b):
        pltpu.make_async_copy(
            rows_v[b], out_hbm.at[pl.ds(0, CHUNK)], osems[b]
        ).wait()

    for b in range(NBUF):
        fire_gather(b, b)

    def group(g0, carry):
        for b in range(NBUF):
            g = g0 * NBUF + b
            wait_gathers(b)
            fire_store(g, b)
            wait_store(b)
            fire_gather(g + NBUF, b)
        return carry

    lax.fori_loop(0, NGROUPS - 1, group, 0)

    for b in range(NBUF):
        g = STEPS - NBUF + b
        wait_gathers(b)
        fire_store(g, b)
    for b in range(NBUF):
        wait_store(b)


def kernel(x, weight):
    flat = x.reshape(-1).astype(jnp.int32)
    out = _embed(flat, weight)
    return out.reshape(B, H, D)


# speedup vs baseline: 1.0022x; 1.0022x over previous
import functools

import jax
import jax.numpy as jnp
from jax import lax
from jax.experimental import pallas as pl
from jax.experimental.pallas import tpu as pltpu
from jax.experimental.pallas import tpu_sc as plsc

B, H, D = 16384, 200, 64
N = B * H
NC, NS = 2, 16
NW = NC * NS
ROWS_W = B // NW
R = 4
CHUNK = R * H
STEPS = ROWS_W // R
NBUF = 2
NGROUPS = STEPS // NBUF

_mesh = plsc.VectorSubcoreMesh(core_axis_name="c", subcore_axis_name="s")


@functools.partial(
    pl.kernel,
    mesh=_mesh,
    out_type=jax.ShapeDtypeStruct((B, H, D), jnp.float32),
    scratch_types=[
        pltpu.VMEM((CHUNK,), jnp.int32),
        pltpu.VMEM((CHUNK,), jnp.int32),
        pltpu.VMEM((R, H, D), jnp.float32),
        pltpu.VMEM((R, H, D), jnp.float32),
        pltpu.SemaphoreType.DMA,
        pltpu.SemaphoreType.DMA,
        pltpu.SemaphoreType.DMA,
        pltpu.SemaphoreType.DMA,
    ],
    compiler_params=pltpu.CompilerParams(use_tc_tiling_on_sc=False),
)
def _embed(idx_hbm, table_hbm, out_hbm, idx0, idx1, rows0, rows1,
           gsem0, gsem1, osem0, osem1):
    idx_v = [idx0, idx1]
    rows_v = [rows0, rows1]
    gsems = [gsem0, gsem1]
    osems = [osem0, osem1]

    wid = lax.axis_index("s") * NC + lax.axis_index("c")
    row_base = wid * ROWS_W

    def fire_gather(chunk_id, b):
        r0 = row_base + chunk_id * R
        pltpu.sync_copy(idx_hbm.at[pl.ds(r0 * H, CHUNK)], idx_v[b])
        for r in range(R):
            pltpu.async_copy(
                table_hbm.at[idx_v[b].at[pl.ds(r * H, H)]],
                rows_v[b].at[r],
                gsems[b],
            )

    def wait_gathers(b):
        pltpu.make_async_copy(
            out_hbm.at[pl.ds(0, R)], rows_v[b], gsems[b]
        ).wait()

    def fire_store(chunk_id, b):
        r0 = row_base + chunk_id * R
        pltpu.async_copy(rows_v[b], out_hbm.at[pl.ds(r0, R)], osems[b])

    def wait_store(b):
        pltpu.make_async_copy(
            rows_v[b], out_hbm.at[pl.ds(0, R)], osems[b]
        ).wait()

    for b in range(NBUF):
        fire_gather(b, b)

    def group(g0, carry):
        for b in range(NBUF):
            g = g0 * NBUF + b
            wait_gathers(b)
            fire_store(g, b)
            wait_store(b)
            fire_gather(g + NBUF, b)
        return carry

    lax.fori_loop(0, NGROUPS - 1, group, 0)

    for b in range(NBUF):
        g = STEPS - NBUF + b
        wait_gathers(b)
        fire_store(g, b)
    for b in range(NBUF):
        wait_store(b)


def kernel(x, weight):
    flat = x.reshape(-1).astype(jnp.int32)
    return _embed(flat, weight)
